# Initial kernel scaffold; baseline (speedup 1.0000x reference)
import jax
import jax.numpy as jnp
from jax.experimental import pallas as pl


def kernel(XA_1d, XB_1d):
    raise NotImplementedError("write your pallas kernel here")


# speedup vs baseline: 72.2733x; 72.2733x over previous
import functools
import math

import jax
import jax.numpy as jnp
from jax import lax
from jax.experimental import pallas as pl
from jax.experimental.pallas import tpu as pltpu
from jax.experimental.pallas import tpu_sc as plsc

N_BINS = 500
PAD_BINS = 512
NC = 2
NS = 16
L = 16
NW = NC * NS
N_ELEM = 16777216
PER_W = N_ELEM // NW
CHUNK = 16384
NCHUNK = PER_W // CHUNK
HIST_WORDS = 3 * PAD_BINS * L
NEG_HALF_LOG_2PI = -0.5 * math.log(2.0 * math.pi)

_mesh = plsc.VectorSubcoreMesh(
    core_axis_name="c", subcore_axis_name="s", num_cores=NC, num_subcores=NS
)


def _wid():
    return lax.axis_index("s") * NC + lax.axis_index("c")


@functools.partial(
    pl.kernel,
    out_type=jax.ShapeDtypeStruct((NW * 64,), jnp.float32),
    mesh=_mesh,
    scratch_types=[
        pltpu.VMEM((CHUNK,), jnp.float32),
        pltpu.VMEM((64,), jnp.float32),
    ],
)
def _minmax_kernel(xa_hbm, xb_hbm, out_hbm, buf, mmv):
    wid = _wid()
    base = wid * PER_W

    big = jnp.full((L,), jnp.inf, jnp.float32)

    def scan_array(x_hbm):
        def chunk_body(i, carry):
            mn, mx = carry
            start = pl.multiple_of(base + i * CHUNK, CHUNK)
            pltpu.sync_copy(x_hbm.at[pl.ds(start, CHUNK)], buf)

            def vec_body(j, c2):
                m, x = c2
                v = buf[pl.ds(j * L, L)]
                return jnp.minimum(m, v), jnp.maximum(x, v)

            return lax.fori_loop(0, CHUNK // L, vec_body, (mn, mx), unroll=8)

        return lax.fori_loop(0, NCHUNK, chunk_body, (big, -big))

    mna, mxa = scan_array(xa_hbm)
    mnb, mxb = scan_array(xb_hbm)

    mmv[pl.ds(0, L)] = mna
    mmv[pl.ds(16, L)] = mxa
    mmv[pl.ds(32, L)] = mnb
    mmv[pl.ds(48, L)] = mxb
    pltpu.sync_copy(mmv, out_hbm.at[pl.ds(wid * 64, 64)])


@functools.partial(
    pl.kernel,
    out_type=jax.ShapeDtypeStruct((NW * HIST_WORDS,), jnp.float32),
    mesh=_mesh,
    scratch_types=[
        pltpu.VMEM((CHUNK,), jnp.float32),
        pltpu.VMEM((HIST_WORDS,), jnp.float32),
        pltpu.VMEM((NW * 64,), jnp.float32),
    ],
    compiler_params=pltpu.CompilerParams(needs_layout_passes=False),
)
def _hist_kernel(xa_hbm, xb_hbm, mm_hbm, out_hbm, buf, hist, mmv):
    def lane_reduce(v, op):
        s = v[0]
        for i in range(1, L):
            s = op(s, v[i])
        return jnp.full((L,), s, jnp.float32)

    wid = _wid()
    base = wid * PER_W

    pltpu.sync_copy(mm_hbm, mmv)

    big = jnp.full((L,), jnp.inf, jnp.float32)

    def fold_body(w, carry):
        mna, mxa, mnb, mxb = carry
        o = w * 64
        return (
            jnp.minimum(mna, mmv[pl.ds(o, L)]),
            jnp.maximum(mxa, mmv[pl.ds(o + 16, L)]),
            jnp.minimum(mnb, mmv[pl.ds(o + 32, L)]),
            jnp.maximum(mxb, mmv[pl.ds(o + 48, L)]),
        )

    mna, mxa, mnb, mxb = lax.fori_loop(0, NW, fold_body, (big, -big, big, -big))

    n_bins_f = jnp.float32(N_BINS)
    one = jnp.full((L,), 1.0, jnp.float32)

    lo_a = lane_reduce(mna, jnp.minimum)
    hi_a = lane_reduce(mxa, jnp.maximum)
    lo_b = lane_reduce(mnb, jnp.minimum)
    hi_b = lane_reduce(mxb, jnp.maximum)
    lo_ab = jnp.minimum(lo_a, lo_b)
    hi_ab = jnp.maximum(hi_a, hi_b)
    inv_a = one / ((hi_a - lo_a) / n_bins_f)
    inv_b = one / ((hi_b - lo_b) / n_bins_f)
    inv_ab = one / ((hi_ab - lo_ab) / n_bins_f)

    zeros = jnp.zeros((L,), jnp.float32)

    def zero_body(i, _):
        hist[pl.ds(i * L, L)] = zeros
        return 0

    lax.fori_loop(0, HIST_WORDS // L, zero_body, 0)

    lane = lax.iota(jnp.int32, L)
    cap = jnp.full((L,), N_BINS - 1, jnp.int32)
    ab_off = 2 * PAD_BINS * L

    def scan_array(x_hbm, lo_own, inv_own, own_off):
        own_base = lane + own_off
        ab_base = lane + ab_off

        def chunk_body(i, _):
            start = pl.multiple_of(base + i * CHUNK, CHUNK)
            pltpu.sync_copy(x_hbm.at[pl.ds(start, CHUNK)], buf)

            def vec_body(j, c2):
                v = buf[pl.ds(j * L, L)]
                io = jnp.minimum(((v - lo_own) * inv_own).astype(jnp.int32), cap)
                ia = jnp.minimum(((v - lo_ab) * inv_ab).astype(jnp.int32), cap)
                plsc.addupdate_scatter(hist, [own_base + io * L], one)
                plsc.addupdate_scatter(hist, [ab_base + ia * L], one)
                return c2

            return lax.fori_loop(0, CHUNK // L, vec_body, 0, unroll=8)

        lax.fori_loop(0, NCHUNK, chunk_body, 0)

    scan_array(xa_hbm, lo_a, inv_a, 0)
    scan_array(xb_hbm, lo_b, inv_b, PAD_BINS * L)

    pltpu.sync_copy(hist, out_hbm.at[pl.ds(wid * HIST_WORDS, HIST_WORDS)])


_ROWS_PER_HIST = PAD_BINS * L // 128
_GRP = 128 // L


def _finalize_body(h_ref, mm_ref, o_ref):
    mm = mm_ref[...]
    lo_a = jnp.min(mm[:, 0:16])
    hi_a = jnp.max(mm[:, 16:32])
    lo_b = jnp.min(mm[:, 32:48])
    hi_b = jnp.max(mm[:, 48:64])
    lo_ab = jnp.minimum(lo_a, lo_b)
    hi_ab = jnp.maximum(hi_a, hi_b)

    h = h_ref[...].reshape(NW, 3 * _ROWS_PER_HIST, 128).sum(axis=0)

    sel = (
        lax.broadcasted_iota(jnp.int32, (128, _GRP), 0) // L
        == lax.broadcasted_iota(jnp.int32, (128, _GRP), 1)
    ).astype(jnp.float32)

    params = [(lo_a, hi_a), (lo_b, hi_b), (lo_ab, hi_ab)]
    terms = []
    for hi_idx, (lo, hi) in enumerate(params):
        block = h[hi_idx * _ROWS_PER_HIST : (hi_idx + 1) * _ROWS_PER_HIST, :]
        counts_mat = jnp.dot(block, sel, preferred_element_type=jnp.float32)
        bins = (
            lax.broadcasted_iota(jnp.int32, (_ROWS_PER_HIST, _GRP), 0) * _GRP
            + lax.broadcasted_iota(jnp.int32, (_ROWS_PER_HIST, _GRP), 1)
        ).astype(jnp.float32)
        width = (hi - lo) / N_BINS
        centers = lo + (bins + 0.5) * width
        lp = -0.5 * centers * centers + NEG_HALF_LOG_2PI
        terms.append(jnp.sum(counts_mat * lp) / jnp.sum(counts_mat))

    log_s = terms[2] - terms[0] - terms[1]
    o_ref[...] = jnp.reshape(log_s, (1, 1))


def kernel(XA_1d, XB_1d):
    mm = _minmax_kernel(XA_1d, XB_1d)
    hists = _hist_kernel(XA_1d, XB_1d, mm)
    out = pl.pallas_call(
        _finalize_body,
        out_shape=jax.ShapeDtypeStruct((1, 1), jnp.float32),
    )(hists.reshape(NW * 3 * _ROWS_PER_HIST, 128), mm.reshape(NW, 64))
    return out[0, 0]


# speedup vs baseline: 290.0354x; 4.0130x over previous
import functools
import math

import jax
import jax.numpy as jnp
from jax import lax
from jax.experimental import pallas as pl
from jax.experimental.pallas import tpu as pltpu
from jax.experimental.pallas import tpu_sc as plsc

N_BINS = 500
PAD_BINS = 512
NC = 2
NS = 16
L = 16
NW = NC * NS
N_ELEM = 16777216
PER_W = N_ELEM // NW
CHUNK = 32768
NCHUNK = PER_W // CHUNK
HIST_WORDS = 3 * PAD_BINS * L
NEG_HALF_LOG_2PI = -0.5 * math.log(2.0 * math.pi)

_mesh = plsc.VectorSubcoreMesh(
    core_axis_name="c", subcore_axis_name="s", num_cores=NC, num_subcores=NS
)


def _wid():
    return lax.axis_index("s") * NC + lax.axis_index("c")


def _splat(val):
    return jnp.where(lax.iota(jnp.int32, L) >= 0, jnp.float32(val), jnp.float32(0))


def _ring_scan(x_hbm, base, buf0, buf1, sem0, sem1, compute, init):
    def start(c, buf, sem):
        s = pl.multiple_of(base + c * CHUNK, CHUNK)
        pltpu.make_async_copy(x_hbm.at[pl.ds(s, CHUNK)], buf, sem).start()

    def wait(buf, sem):
        pltpu.make_async_copy(x_hbm.at[pl.ds(0, CHUNK)], buf, sem).wait()

    start(0, buf0, sem0)
    start(1, buf1, sem1)

    def body(k, carry):
        wait(buf0, sem0)
        carry = compute(buf0, carry)

        @pl.when(2 * k + 2 < NCHUNK)
        def _s0():
            start(2 * k + 2, buf0, sem0)

        wait(buf1, sem1)
        carry = compute(buf1, carry)

        @pl.when(2 * k + 3 < NCHUNK)
        def _s1():
            start(2 * k + 3, buf1, sem1)

        return carry

    return lax.fori_loop(0, NCHUNK // 2, body, init)


@functools.partial(
    pl.kernel,
    out_type=jax.ShapeDtypeStruct((NW * 64,), jnp.float32),
    mesh=_mesh,
    scratch_types=[
        pltpu.VMEM((CHUNK,), jnp.float32),
        pltpu.VMEM((CHUNK,), jnp.float32),
        pltpu.VMEM((64,), jnp.float32),
        pltpu.SemaphoreType.DMA,
        pltpu.SemaphoreType.DMA,
    ],
)
def _minmax_kernel(xa_hbm, xb_hbm, out_hbm, buf0, buf1, mmv, sem0, sem1):
    wid = _wid()
    base = wid * PER_W

    big = _splat(jnp.inf)
    nacc = 8
    nvec8 = CHUNK // L // nacc

    def scan_array(x_hbm):
        def compute(buf, carry):
            def body8(i, c2):
                mns, mxs = c2
                mns, mxs = list(mns), list(mxs)
                for u in range(nacc):
                    v = buf[pl.ds((i * nacc + u) * L, L)]
                    mns[u] = jnp.minimum(mns[u], v)
                    mxs[u] = jnp.maximum(mxs[u], v)
                return tuple(mns), tuple(mxs)

            return lax.fori_loop(0, nvec8, body8, carry)

        mns, mxs = _ring_scan(
            x_hbm, base, buf0, buf1, sem0, sem1, compute,
            ((big,) * nacc, (-big,) * nacc),
        )
        mn = functools.reduce(jnp.minimum, mns)
        mx = functools.reduce(jnp.maximum, mxs)
        return mn, mx

    mna, mxa = scan_array(xa_hbm)
    mnb, mxb = scan_array(xb_hbm)

    mmv[pl.ds(0, L)] = mna
    mmv[pl.ds(16, L)] = mxa
    mmv[pl.ds(32, L)] = mnb
    mmv[pl.ds(48, L)] = mxb
    pltpu.sync_copy(mmv, out_hbm.at[pl.ds(wid * 64, 64)])


@functools.partial(
    pl.kernel,
    out_type=jax.ShapeDtypeStruct((NW * HIST_WORDS,), jnp.float32),
    mesh=_mesh,
    scratch_types=[
        pltpu.VMEM((CHUNK,), jnp.float32),
        pltpu.VMEM((CHUNK,), jnp.float32),
        pltpu.VMEM((HIST_WORDS,), jnp.float32),
        pltpu.VMEM((NW * 64,), jnp.float32),
        pltpu.SemaphoreType.DMA,
        pltpu.SemaphoreType.DMA,
    ],
    compiler_params=pltpu.CompilerParams(needs_layout_passes=False),
)
def _hist_kernel(xa_hbm, xb_hbm, mm_hbm, out_hbm, buf0, buf1, hist, mmv, sem0, sem1):
    def lane_reduce(v, op):
        s = v[0]
        for i in range(1, L):
            s = op(s, v[i])
        return jnp.full((L,), s, jnp.float32)

    wid = _wid()
    base = wid * PER_W

    pltpu.sync_copy(mm_hbm, mmv)

    big = _splat(jnp.inf)

    def fold_body(w, carry):
        mna, mxa, mnb, mxb = carry
        o = w * 64
        return (
            jnp.minimum(mna, mmv[pl.ds(o, L)]),
            jnp.maximum(mxa, mmv[pl.ds(o + 16, L)]),
            jnp.minimum(mnb, mmv[pl.ds(o + 32, L)]),
            jnp.maximum(mxb, mmv[pl.ds(o + 48, L)]),
        )

    mna, mxa, mnb, mxb = lax.fori_loop(0, NW, fold_body, (big, -big, big, -big))

    n_bins_f = jnp.float32(N_BINS)
    one = _splat(1.0)

    lo_a = lane_reduce(mna, jnp.minimum)
    hi_a = lane_reduce(mxa, jnp.maximum)
    lo_b = lane_reduce(mnb, jnp.minimum)
    hi_b = lane_reduce(mxb, jnp.maximum)
    lo_ab = jnp.minimum(lo_a, lo_b)
    hi_ab = jnp.maximum(hi_a, hi_b)
    inv_a = one / ((hi_a - lo_a) / n_bins_f)
    inv_b = one / ((hi_b - lo_b) / n_bins_f)
    inv_ab = one / ((hi_ab - lo_ab) / n_bins_f)

    zeros = _splat(0.0)

    def zero_body(i, _):
        hist[pl.ds(i * L, L)] = zeros
        return 0

    lax.fori_loop(0, HIST_WORDS // L, zero_body, 0)

    lane = lax.iota(jnp.int32, L)
    ab_lanes = lane + 2 * PAD_BINS * L

    def scan_array(x_hbm, lo_own, inv_own, own_off):
        own_lanes = lane + own_off

        def compute(buf, carry):
            def body(j):
                v = buf[pl.ds(j * L, L)]
                io = ((v - lo_own) * inv_own).astype(jnp.int32)
                ia = ((v - lo_ab) * inv_ab).astype(jnp.int32)
                plsc.addupdate_scatter(hist, [(io << 4) | own_lanes], one)
                plsc.addupdate_scatter(hist, [(ia << 4) | ab_lanes], one)

            plsc.parallel_loop(0, CHUNK // L, unroll=8)(body)
            return carry

        _ring_scan(x_hbm, base, buf0, buf1, sem0, sem1, compute, 0)

    scan_array(xa_hbm, lo_a, inv_a, 0)
    scan_array(xb_hbm, lo_b, inv_b, PAD_BINS * L)

    pltpu.sync_copy(hist, out_hbm.at[pl.ds(wid * HIST_WORDS, HIST_WORDS)])


_ROWS_PER_HIST = PAD_BINS * L // 128
_GRP = 128 // L


def _finalize_body(h_ref, mm_ref, o_ref):
    mm = mm_ref[...]
    lo_a = jnp.min(mm[:, 0:16])
    hi_a = jnp.max(mm[:, 16:32])
    lo_b = jnp.min(mm[:, 32:48])
    hi_b = jnp.max(mm[:, 48:64])
    lo_ab = jnp.minimum(lo_a, lo_b)
    hi_ab = jnp.maximum(hi_a, hi_b)

    h = h_ref[...].reshape(NW, 3 * _ROWS_PER_HIST, 128).sum(axis=0)

    sel = (
        lax.broadcasted_iota(jnp.int32, (128, _GRP), 0) // L
        == lax.broadcasted_iota(jnp.int32, (128, _GRP), 1)
    ).astype(jnp.float32)

    params = [(lo_a, hi_a), (lo_b, hi_b), (lo_ab, hi_ab)]
    terms = []
    for hi_idx, (lo, hi) in enumerate(params):
        block = h[hi_idx * _ROWS_PER_HIST : (hi_idx + 1) * _ROWS_PER_HIST, :]
        counts_mat = jnp.dot(block, sel, preferred_element_type=jnp.float32)
        bins_i = (
            lax.broadcasted_iota(jnp.int32, (_ROWS_PER_HIST, _GRP), 0) * _GRP
            + lax.broadcasted_iota(jnp.int32, (_ROWS_PER_HIST, _GRP), 1)
        )
        bins = bins_i.astype(jnp.float32)
        width = (hi - lo) / N_BINS
        centers = lo + (bins + 0.5) * width
        lp = -0.5 * centers * centers + NEG_HALF_LOG_2PI
        c499 = lo + (N_BINS - 0.5) * width
        lp499 = -0.5 * c499 * c499 + NEG_HALF_LOG_2PI
        lp_eff = jnp.where(bins_i >= N_BINS - 1, lp499, lp)
        terms.append(jnp.sum(counts_mat * lp_eff) / jnp.sum(counts_mat))

    log_s = terms[2] - terms[0] - terms[1]
    o_ref[...] = jnp.reshape(log_s, (1, 1))


def kernel(XA_1d, XB_1d):
    mm = _minmax_kernel(XA_1d, XB_1d)
    hists = _hist_kernel(XA_1d, XB_1d, mm)
    out = pl.pallas_call(
        _finalize_body,
        out_shape=jax.ShapeDtypeStruct((1, 1), jnp.float32),
    )(hists.reshape(NW * 3 * _ROWS_PER_HIST, 128), mm.reshape(NW, 64))
    return out[0, 0]
